# baseline (speedup 1.0000x reference)
import functools

import jax
import jax.numpy as jnp
from jax import lax
from jax.experimental import pallas as pl
from jax.experimental.pallas import tpu as pltpu

_N_BINS = 25
_BIN_PAD = 32


def _ece_body(logits_ref, labels_ref, out_ref, stats_ref, *, n_total, num_blocks):
    i = pl.program_id(0)

    @pl.when(i == 0)
    def _init():
        stats_ref[...] = jnp.zeros_like(stats_ref)

    x = logits_ref[...]
    bn, c = x.shape
    labels = labels_ref[0]

    rowmax = jnp.max(x, axis=1, keepdims=True)
    sumexp = jnp.sum(jnp.exp(x - rowmax), axis=1, keepdims=True)
    conf = 1.0 / sumexp

    class_iota = lax.broadcasted_iota(jnp.int32, (bn, c), 1)
    pred = jnp.min(jnp.where(x == rowmax, class_iota, c), axis=1, keepdims=True)
    acc = (pred == labels).astype(jnp.float32)

    delta = jnp.float32(1.0 / _N_BINS)
    bin_iota = lax.broadcasted_iota(jnp.int32, (bn, _BIN_PAD), 1).astype(jnp.float32)
    lo = bin_iota * delta
    hi = (bin_iota + 1.0) * delta
    in_bin = ((conf > lo) & (conf <= hi)).astype(jnp.float32)

    count_part = jnp.sum(in_bin, axis=0, keepdims=True)
    csum_part = jnp.sum(conf * in_bin, axis=0, keepdims=True)
    asum_part = jnp.sum(acc * in_bin, axis=0, keepdims=True)

    stats_ref[0:1, 0:_BIN_PAD] += count_part
    stats_ref[1:2, 0:_BIN_PAD] += csum_part
    stats_ref[2:3, 0:_BIN_PAD] += asum_part

    @pl.when(i == num_blocks - 1)
    def _finish():
        count = stats_ref[0:1, 0:_BIN_PAD]
        csum = stats_ref[1:2, 0:_BIN_PAD]
        asum = stats_ref[2:3, 0:_BIN_PAD]
        safe = jnp.maximum(count, 1.0)
        gaps = jnp.where(
            count > 0.0,
            jnp.abs(csum / safe - asum / safe) * (count / n_total),
            0.0,
        )
        out_ref[...] = jnp.sum(gaps, axis=1, keepdims=True)


def kernel(logits, labels):
    n, c = logits.shape
    block_n = min(1024, n)
    num_blocks = n // block_n
    labels3 = labels.reshape(num_blocks, block_n, 1)

    body = functools.partial(
        _ece_body, n_total=float(n), num_blocks=num_blocks
    )
    out = pl.pallas_call(
        body,
        grid=(num_blocks,),
        in_specs=[
            pl.BlockSpec((block_n, c), lambda i: (i, 0)),
            pl.BlockSpec((1, block_n, 1), lambda i: (i, 0, 0)),
        ],
        out_specs=pl.BlockSpec((1, 1), lambda i: (0, 0)),
        out_shape=jax.ShapeDtypeStruct((1, 1), jnp.float32),
        scratch_shapes=[pltpu.VMEM((8, 128), jnp.float32)],
    )(logits, labels3)
    return out.reshape(1)
